# Initial kernel scaffold; baseline (speedup 1.0000x reference)
import jax
import jax.numpy as jnp
from jax.experimental import pallas as pl


def kernel(tokens, emb):
    raise NotImplementedError("write your pallas kernel here")


# speedup vs baseline: 4.1960x; 4.1960x over previous
import functools

import jax
import jax.numpy as jnp
from jax import lax
from jax.experimental import pallas as pl
from jax.experimental.pallas import tpu as pltpu
from jax.experimental.pallas import tpu_sc as plsc

BATCH = 4096
SEQ = 200
D_EMBED = 32
TOTAL = BATCH * SEQ
NUM_WORKERS = 32
PER_WORKER = TOTAL // NUM_WORKERS
CHUNK = 128
NCHUNKS = PER_WORKER // CHUNK


def _sc_gather(tokens2d, emb):
    mesh = plsc.VectorSubcoreMesh(core_axis_name="c", subcore_axis_name="s")

    @functools.partial(
        pl.kernel,
        mesh=mesh,
        out_type=jax.ShapeDtypeStruct((TOTAL, D_EMBED), jnp.float32),
        scratch_types=[
            pltpu.VMEM((NCHUNKS, CHUNK), jnp.int32),
            pltpu.VMEM((CHUNK, D_EMBED), jnp.float32),
            pltpu.SemaphoreType.DMA,
        ],
        compiler_params=pltpu.CompilerParams(use_tc_tiling_on_sc=False),
    )
    def k(tok_hbm, emb_hbm, out_hbm, idx_v, rows_v, sem):
        wid = lax.axis_index("s") * 2 + lax.axis_index("c")
        pltpu.sync_copy(tok_hbm.at[pl.ds(wid * NCHUNKS, NCHUNKS)], idx_v)
        out_base = wid * PER_WORKER

        def body(j, carry):
            pltpu.async_copy(emb_hbm.at[idx_v.at[j]], rows_v, sem).wait()
            pltpu.sync_copy(rows_v, out_hbm.at[pl.ds(out_base + j * CHUNK, CHUNK)])
            return carry

        lax.fori_loop(0, NCHUNKS, body, 0)

    return k(tokens2d, emb)


def kernel(tokens, emb):
    tok2d = tokens.reshape(TOTAL // CHUNK, CHUNK).astype(jnp.int32)
    out = _sc_gather(tok2d, emb)
    return out.reshape(BATCH, SEQ, D_EMBED)


# speedup vs baseline: 5.2573x; 1.2529x over previous
import functools

import jax
import jax.numpy as jnp
from jax import lax
from jax.experimental import pallas as pl
from jax.experimental.pallas import tpu as pltpu
from jax.experimental.pallas import tpu_sc as plsc

BATCH = 4096
SEQ = 200
D_EMBED = 32
TOTAL = BATCH * SEQ
NUM_WORKERS = 32
PER_WORKER = TOTAL // NUM_WORKERS
CHUNK = 128
NCHUNKS = PER_WORKER // CHUNK
K = 8
NROUNDS = NCHUNKS // K


def _sc_gather(tokens2d, emb):
    mesh = plsc.VectorSubcoreMesh(core_axis_name="c", subcore_axis_name="s")

    @functools.partial(
        pl.kernel,
        mesh=mesh,
        out_type=jax.ShapeDtypeStruct((TOTAL, D_EMBED), jnp.float32),
        scratch_types=[
            pltpu.VMEM((NCHUNKS, CHUNK), jnp.int32),
            pltpu.VMEM((K, CHUNK, D_EMBED), jnp.float32),
            pltpu.SemaphoreType.DMA((K,)),
            pltpu.SemaphoreType.DMA((K,)),
        ],
        compiler_params=pltpu.CompilerParams(use_tc_tiling_on_sc=False),
    )
    def k(tok_hbm, emb_hbm, out_hbm, idx_v, rows_v, gsem, ssem):
        wid = lax.axis_index("s") * 2 + lax.axis_index("c")
        pltpu.sync_copy(tok_hbm.at[pl.ds(wid * NCHUNKS, NCHUNKS)], idx_v)
        out_base = wid * PER_WORKER

        def round_body(g, carry):
            base = g * K
            gathers = []
            for b in range(K):
                gathers.append(pltpu.async_copy(
                    emb_hbm.at[idx_v.at[base + b]], rows_v.at[b], gsem.at[b]))
            stores = []
            for b in range(K):
                gathers[b].wait()
                stores.append(pltpu.async_copy(
                    rows_v.at[b],
                    out_hbm.at[pl.ds(out_base + (base + b) * CHUNK, CHUNK)],
                    ssem.at[b]))
            for b in range(K):
                stores[b].wait()
            return carry

        lax.fori_loop(0, NROUNDS, round_body, 0)

    return k(tokens2d, emb)


def kernel(tokens, emb):
    tok2d = tokens.reshape(TOTAL // CHUNK, CHUNK).astype(jnp.int32)
    out = _sc_gather(tok2d, emb)
    return out.reshape(BATCH, SEQ, D_EMBED)
